# baseline (speedup 1.0000x reference)
import functools, jax, jax.numpy as jnp
from jax import lax
from jax.experimental import pallas as pl
from jax.experimental.pallas import tpu as pltpu
from jax.experimental.pallas import tpu_sc as plsc


def kernel(inp, table, W1, b1):
    mesh = plsc.VectorSubcoreMesh(core_axis_name="c", subcore_axis_name="s")

    @functools.partial(
        pl.kernel, mesh=mesh,
        out_type=jax.ShapeDtypeStruct((8, 16), jnp.float32),
        scratch_types=[pltpu.VMEM((8, 16), jnp.float32)],
    )
    def _sc_copy(b_hbm, out_hbm, v):
        @pl.when((lax.axis_index("c") == 0) & (lax.axis_index("s") == 0))
        def _():
            pltpu.sync_copy(b_hbm, v)
            pltpu.sync_copy(v, out_hbm)

    small = b1[:128].reshape(8, 16)
    return _sc_copy(small)


# speedup vs baseline: 6.5712x; 6.5712x over previous
import jax
import jax.numpy as jnp
from jax.experimental import pallas as pl
from jax.experimental.pallas import tpu as pltpu

VOCAB = 100000
EMB = 64
CTX = 3
HID = 512
NTOK = 2 * CTX
LANES = 128


def _cbow_body(idx_ref, *refs):
    blk_refs = refs[:NTOK]
    w1_ref, b1_ref, out_ref = refs[NTOK:]
    lane_ids = jax.lax.broadcasted_iota(jnp.int32, (1, LANES), 1)
    rows = []
    for t in range(NTOK):
        onehot = (lane_ids == idx_ref[t] % LANES).astype(jnp.float32)
        valid = (idx_ref[t] // LANES) * LANES + lane_ids < VOCAB
        blk = jnp.where(valid, blk_refs[t][...], 0.0)
        rows.append(jax.lax.dot_general(
            onehot, blk, (((1,), (1,)), ((), ())),
            preferred_element_type=jnp.float32))
    h = jnp.concatenate(rows, axis=1)
    logits = jax.lax.dot_general(
        h, w1_ref[...], (((1,), (1,)), ((), ())),
        preferred_element_type=jnp.float32)
    logits = jnp.maximum(logits + b1_ref[...], 0.0)
    m = jnp.max(logits, axis=1, keepdims=True)
    lse = jnp.log(jnp.sum(jnp.exp(logits - m), axis=1, keepdims=True)) + m
    out_ref[...] = logits - lse


def _blk_spec(t):
    return pl.BlockSpec(
        (EMB, LANES), lambda i, idx_ref, t=t: (0, idx_ref[t] // LANES))


def kernel(inp, table, W1, b1):
    idx = inp.astype(jnp.int32)
    b1r = b1.reshape(1, HID)
    tableT = table.T
    grid_spec = pltpu.PrefetchScalarGridSpec(
        num_scalar_prefetch=1,
        grid=(1,),
        in_specs=[_blk_spec(t) for t in range(NTOK)] + [
            pl.BlockSpec((HID, NTOK * EMB), lambda i, idx_ref: (0, 0)),
            pl.BlockSpec((1, HID), lambda i, idx_ref: (0, 0)),
        ],
        out_specs=pl.BlockSpec((1, HID), lambda i, idx_ref: (0, 0)),
    )
    return pl.pallas_call(
        _cbow_body,
        grid_spec=grid_spec,
        out_shape=jax.ShapeDtypeStruct((1, HID), jnp.float32),
    )(idx, *([tableT] * NTOK), W1, b1r)
